# Initial kernel scaffold; baseline (speedup 1.0000x reference)
import jax
import jax.numpy as jnp
from jax.experimental import pallas as pl


def kernel(entity_embed, edge_index, edge_type, rel_embed, W_R, W_w, W_b):
    raise NotImplementedError("write your pallas kernel here")


# speedup vs baseline: 13.6039x; 13.6039x over previous
import functools

import jax
import jax.numpy as jnp
from jax import lax
from jax.experimental import pallas as pl
from jax.experimental.pallas import tpu as pltpu
from jax.experimental.pallas import tpu_sc as plsc

N = 10000
D = 128
R = 8
E = 320000

NB = 1000
NBLK = N // NB

NC = 2
NS = 16
NW = NC * NS
CHUNK = 128
CPW = 79
EPW = CPW * CHUNK
E_PAD = EPW * NW
DUMP = N
RPT = 632
N_ACC = RPT * NS


def _xscaled_body(emb_ref, wr_ref, remb_ref, out_ref):
    emb = emb_ref[...]
    x = jnp.dot(emb, wr_ref[0], preferred_element_type=jnp.float32)
    att = jnp.sum(x * jnp.tanh(x + remb_ref[0]), axis=1, keepdims=True)
    out_ref[...] = emb * att


def _make_xscaled(emb, W_R, rel_embed):
    return pl.pallas_call(
        _xscaled_body,
        grid=(NBLK, R),
        in_specs=[
            pl.BlockSpec((NB, D), lambda i, r: (i, 0)),
            pl.BlockSpec((1, D, D), lambda i, r: (r, 0, 0)),
            pl.BlockSpec((1, 1, D), lambda i, r: (r, 0, 0)),
        ],
        out_specs=pl.BlockSpec((NB, D), lambda i, r: (r * NBLK + i, 0)),
        out_shape=jax.ShapeDtypeStruct((R * N, D), jnp.float32),
    )(emb, W_R, rel_embed.reshape(R, 1, D))


_SC_MESH = plsc.VectorSubcoreMesh(core_axis_name="c", subcore_axis_name="s")


@functools.partial(
    pl.kernel,
    mesh=_SC_MESH,
    out_type=jax.ShapeDtypeStruct((NC * N_ACC, D), jnp.float32),
    scratch_types=[
        pltpu.VMEM((CHUNK,), jnp.int32),
        pltpu.VMEM((CHUNK,), jnp.int32),
        pltpu.VMEM((CHUNK,), jnp.int32),
        pltpu.VMEM((CHUNK,), jnp.int32),
        pltpu.VMEM((CHUNK, D), jnp.float32),
        pltpu.VMEM((8, D), jnp.float32),
        pltpu.VMEM_SHARED((N_ACC, D), jnp.float32),
        pltpu.SemaphoreType.DMA,
    ],
)
def _sc_aggregate(xs_hbm, src_hbm, et_hbm, dst_hbm, out_hbm,
                  sbuf, tbuf, dbuf, aidx, rows, stage, acc, sem):
    c = lax.axis_index("c")
    s = lax.axis_index("s")
    wid = s * NC + c

    zv = jnp.zeros((16,), jnp.float32)
    for i in range(8):
        for j in range(8):
            stage[i, pl.ds(j * 16, 16)] = zv

    def _zero(k, carry):
        pltpu.sync_copy(stage, acc.at[pl.ds(s * RPT + k * 8, 8)])
        return carry

    lax.fori_loop(0, RPT // 8, _zero, 0)
    plsc.subcore_barrier()

    def _chunk(ci, carry):
        base = wid * EPW + ci * CHUNK
        pltpu.sync_copy(src_hbm.at[pl.ds(base, CHUNK)], sbuf)
        pltpu.sync_copy(et_hbm.at[pl.ds(base, CHUNK)], tbuf)
        pltpu.sync_copy(dst_hbm.at[pl.ds(base, CHUNK)], dbuf)
        for j in range(CHUNK // 16):
            sl = pl.ds(j * 16, 16)
            aidx[sl] = tbuf[sl] * N + sbuf[sl]
        pltpu.async_copy(xs_hbm.at[aidx], rows, sem).wait()
        pltpu.sync_copy(rows, acc.at[dbuf], add=True)
        return carry

    lax.fori_loop(0, CPW, _chunk, 0)
    plsc.subcore_barrier()

    def _export(k, carry):
        r0 = s * RPT + k * 8
        pltpu.sync_copy(acc.at[pl.ds(r0, 8)], stage)
        pltpu.sync_copy(stage, out_hbm.at[pl.ds(c * N_ACC + r0, 8)])
        return carry

    lax.fori_loop(0, RPT // 8, _export, 0)


def _out_body(emb_ref, p0_ref, p1_ref, w1_ref, w2_ref, b_ref, out_ref):
    emb = emb_ref[...]
    nf = p0_ref[...] + p1_ref[...]
    h = (jnp.dot(emb, w1_ref[...], preferred_element_type=jnp.float32)
         + jnp.dot(nf, w2_ref[...], preferred_element_type=jnp.float32)
         + b_ref[...])
    out_ref[...] = jnp.where(h >= 0, h, 0.01 * h) + emb


def _make_out(emb, p0, p1, w1t, w2t, b):
    return pl.pallas_call(
        _out_body,
        grid=(NBLK,),
        in_specs=[
            pl.BlockSpec((NB, D), lambda i: (i, 0)),
            pl.BlockSpec((NB, D), lambda i: (i, 0)),
            pl.BlockSpec((NB, D), lambda i: (i, 0)),
            pl.BlockSpec((D, D), lambda i: (0, 0)),
            pl.BlockSpec((D, D), lambda i: (0, 0)),
            pl.BlockSpec((1, D), lambda i: (0, 0)),
        ],
        out_specs=pl.BlockSpec((NB, D), lambda i: (i, 0)),
        out_shape=jax.ShapeDtypeStruct((N, D), jnp.float32),
    )(emb, p0, p1, w1t, w2t, b)


def kernel(entity_embed, edge_index, edge_type, rel_embed, W_R, W_w, W_b):
    emb = entity_embed.astype(jnp.float32)
    src = edge_index[0].astype(jnp.int32)
    dst = edge_index[1].astype(jnp.int32)
    et = edge_type.astype(jnp.int32)

    pad = E_PAD - E
    src = jnp.concatenate([src, jnp.zeros((pad,), jnp.int32)])
    dst = jnp.concatenate([dst, jnp.full((pad,), DUMP, jnp.int32)])
    et = jnp.concatenate([et, jnp.zeros((pad,), jnp.int32)])

    xs = _make_xscaled(emb, W_R.astype(jnp.float32), rel_embed.astype(jnp.float32))
    partials = _sc_aggregate(xs, src, et, dst)
    p0 = partials[:N]
    p1 = partials[N_ACC:N_ACC + N]

    w1t = W_w[:, :D].T
    w2t = W_w[:, D:].T
    return _make_out(emb, p0, p1, w1t, w2t, W_b.reshape(1, D))
